# Initial kernel scaffold; baseline (speedup 1.0000x reference)
import jax
import jax.numpy as jnp
from jax.experimental import pallas as pl


def kernel(query_feat, context_feats, offset_w1, offset_b1, offset_w2, offset_b2, attn_w1, attn_b1, attn_w2, attn_b2, v_w, v_b, out_w, out_b):
    raise NotImplementedError("write your pallas kernel here")


# speedup vs baseline: 2.0572x; 2.0572x over previous
import functools

import jax
import jax.numpy as jnp
from jax import lax
from jax.experimental import pallas as pl
from jax.experimental.pallas import tpu as pltpu
from jax.experimental.pallas import tpu_sc as plsc

_B, _C, _H, _W = 1, 96, 64, 64
_NH, _K, _T = 4, 9, 7
_HD = _C // _NH
_TK = _T * _K
_NP = _H * _W
_MAX_OFFSET = 32.0

_TILE = 512
_NT = _NP // _TILE
_BLK = 256
_NBLK = _NP // _BLK
_ROWS = _NH * _TK


def _gelu(x):
    return 0.5 * x * (1.0 + lax.erf(x * 0.7071067811865476))


def _stage_a(x9_ref, ctx_ref, wo1_ref, bo1_ref, wox_ref, box_ref, woy_ref,
             boy_ref, wa1_ref, ba1_ref, wa2_ref, ba2_ref, vw_ref, vb_ref,
             vt_ref, w00_ref, w01_ref, w10_ref, w11_ref,
             i00_ref, i01_ref, i10_ref, i11_ref):
    f32 = jnp.float32
    i = pl.program_id(0)
    x9 = x9_ref[...]

    t1 = _gelu(jnp.dot(wo1_ref[...], x9, preferred_element_type=f32)
               + bo1_ref[...])
    offx = jnp.tanh(jnp.dot(wox_ref[...], t1, preferred_element_type=f32)
                    + box_ref[...]) * _MAX_OFFSET
    offy = jnp.tanh(jnp.dot(woy_ref[...], t1, preferred_element_type=f32)
                    + boy_ref[...]) * _MAX_OFFSET

    a1 = _gelu(jnp.dot(wa1_ref[...], x9, preferred_element_type=f32)
               + ba1_ref[...])
    al = jnp.dot(wa2_ref[...], a1, preferred_element_type=f32) + ba2_ref[...]
    aw_parts = []
    for h in range(_NH):
        alh = al[h * _TK:(h + 1) * _TK, :]
        m = jnp.max(alh, axis=0, keepdims=True)
        e = jnp.exp(alh - m)
        aw_parts.append(e / jnp.sum(e, axis=0, keepdims=True))
    aw = jnp.concatenate(aw_parts, axis=0)

    for t in range(_T):
        vt_ref[t] = (jnp.dot(vw_ref[...], ctx_ref[t],
                             preferred_element_type=f32) + vb_ref[...])

    ii = lax.broadcasted_iota(jnp.int32, (1, _TILE), 1)
    gxf = (ii & (_W - 1)).astype(f32)
    gyf = ((ii >> 6) + i * (_TILE // _W)).astype(f32)

    sgx = ((gxf + offx) / (_W - 1)) * 2.0 - 1.0
    sgy = ((gyf + offy) / (_H - 1)) * 2.0 - 1.0
    gxr = (sgx + 1.0) * 0.5 * (_W - 1)
    gyr = (sgy + 1.0) * 0.5 * (_H - 1)

    x0 = jnp.floor(gxr)
    y0 = jnp.floor(gyr)
    wx1 = gxr - x0
    wx0 = 1.0 - wx1
    wy1 = gyr - y0
    wy0 = 1.0 - wy1
    x1 = x0 + 1.0
    y1 = y0 + 1.0

    def corner(xf, yf, wgt):
        valid = ((xf >= 0.0) & (xf <= _W - 1.0)
                 & (yf >= 0.0) & (yf <= _H - 1.0)).astype(f32)
        xi = jnp.clip(xf, 0.0, _W - 1.0).astype(jnp.int32)
        yi = jnp.clip(yf, 0.0, _H - 1.0).astype(jnp.int32)
        return aw * wgt * valid, yi * _W + xi

    w00_ref[...], i00_ref[...] = corner(x0, y0, wy0 * wx0)
    w01_ref[...], i01_ref[...] = corner(x1, y0, wy0 * wx1)
    w10_ref[...], i10_ref[...] = corner(x0, y1, wy1 * wx0)
    w11_ref[...], i11_ref[...] = corner(x1, y1, wy1 * wx1)


def _stage_b(w00, w01, w10, w11, i00, i01, i10, i11, tab, out,
             tab_v, w_v, idx_v, acc_v):
    f32 = jnp.float32
    wid = lax.axis_index("s") * 2 + lax.axis_index("c")
    h = wid // _T
    t = wid - h * _T

    @pl.when(wid < _NH * _T)
    def _():
        pltpu.sync_copy(tab.at[t, pl.ds(h * _HD, _HD), :], tab_v)
        row0 = h * _TK + t * _K

        def blk_body(b, carry):
            col = b * _BLK
            for ci, (wr, ir) in enumerate(((w00, i00), (w01, i01),
                                           (w10, i10), (w11, i11))):
                pltpu.sync_copy(wr.at[pl.ds(row0, _K), pl.ds(col, _BLK)],
                                w_v.at[pl.ds(ci * _K, _K), :])
                pltpu.sync_copy(ir.at[pl.ds(row0, _K), pl.ds(col, _BLK)],
                                idx_v.at[pl.ds(ci * _K, _K), :])

            zvec = jnp.zeros((16,), f32)

            def zero_body(z, cz):
                acc_v[z >> 4, pl.ds((z & 15) * 16, 16)] = zvec
                return cz

            lax.fori_loop(0, _HD * _BLK // 16, zero_body, 0)

            def pg_body(pg, c2):
                base = pg * 16

                def j_body(j, cj):
                    idxv = idx_v[j, pl.ds(base, 16)]
                    wv = w_v[j, pl.ds(base, 16)]

                    def c_body(c, cc):
                        cvec = jnp.full((16,), 0, jnp.int32) + c
                        g = plsc.load_gather(tab_v, [cvec, idxv])
                        plsc.addupdate(acc_v.at[c, pl.ds(base, 16)], wv * g)
                        return cc

                    lax.fori_loop(0, _HD, c_body, 0, unroll=8)
                    return cj

                lax.fori_loop(0, 4 * _K, j_body, 0)
                return c2

            lax.fori_loop(0, _BLK // 16, pg_body, 0)
            pltpu.sync_copy(acc_v, out.at[t, h, :, pl.ds(col, _BLK)])
            return carry

        lax.fori_loop(0, _NBLK, blk_body, 0, unroll=False)


def _stage_c(part_ref, q_ref, wout_ref, bout_ref, out_ref):
    s = part_ref[0]
    for t in range(1, _T):
        s = s + part_ref[t]
    out_ref[...] = (jnp.dot(wout_ref[...], s, preferred_element_type=jnp.float32)
                    + bout_ref[...] + q_ref[...])


def _full(shape):
    return pl.BlockSpec(shape, lambda i: tuple(0 for _ in shape))


def kernel(query_feat, context_feats, offset_w1, offset_b1, offset_w2,
           offset_b2, attn_w1, attn_b1, attn_w2, attn_b2, v_w, v_b,
           out_w, out_b):
    f32 = jnp.float32
    i32 = jnp.int32

    q2 = query_feat.reshape(_C, _H, _W)
    qp = jnp.pad(q2, ((0, 0), (1, 1), (1, 1)))
    shifts = [qp[:, dy:dy + _H, dx:dx + _W].reshape(_C, _NP)
              for dy in range(3) for dx in range(3)]
    x9 = jnp.concatenate(shifts, axis=0)

    wo1 = jnp.transpose(offset_w1, (0, 2, 3, 1)).reshape(_C, 9 * _C)
    wa1 = jnp.transpose(attn_w1, (0, 2, 3, 1)).reshape(_C, 9 * _C)
    w2r = offset_w2.reshape(_NH, _T, _K, 2, _C)
    wox = w2r[:, :, :, 0, :].reshape(_ROWS, _C)
    woy = w2r[:, :, :, 1, :].reshape(_ROWS, _C)
    b2r = offset_b2.reshape(_NH, _T, _K, 2)
    box = b2r[..., 0].reshape(_ROWS, 1)
    boy = b2r[..., 1].reshape(_ROWS, 1)
    wa2 = attn_w2.reshape(_ROWS, _C)
    ba2 = attn_b2.reshape(_ROWS, 1)
    bo1 = offset_b1.reshape(_C, 1)
    ba1 = attn_b1.reshape(_C, 1)
    vw2 = v_w.reshape(_C, _C)
    vb2 = v_b.reshape(_C, 1)
    ctx = context_feats.reshape(_T, _C, _NP)

    plane_f = jax.ShapeDtypeStruct((_ROWS, _NP), f32)
    plane_i = jax.ShapeDtypeStruct((_ROWS, _NP), i32)
    a_out = pl.pallas_call(
        _stage_a,
        grid=(_NT,),
        in_specs=[
            pl.BlockSpec((9 * _C, _TILE), lambda i: (0, i)),
            pl.BlockSpec((_T, _C, _TILE), lambda i: (0, 0, i)),
            _full((_C, 9 * _C)), _full((_C, 1)),
            _full((_ROWS, _C)), _full((_ROWS, 1)),
            _full((_ROWS, _C)), _full((_ROWS, 1)),
            _full((_C, 9 * _C)), _full((_C, 1)),
            _full((_ROWS, _C)), _full((_ROWS, 1)),
            _full((_C, _C)), _full((_C, 1)),
        ],
        out_specs=[
            pl.BlockSpec((_T, _C, _TILE), lambda i: (0, 0, i)),
        ] + [pl.BlockSpec((_ROWS, _TILE), lambda i: (0, i))] * 8,
        out_shape=[jax.ShapeDtypeStruct((_T, _C, _NP), f32)]
        + [plane_f] * 4 + [plane_i] * 4,
    )(x9, ctx, wo1, bo1, wox, box, woy, boy, wa1, ba1, wa2, ba2, vw2, vb2)
    vt, w00, w01, w10, w11, idx00, idx01, idx10, idx11 = a_out

    sc_call = pl.kernel(
        _stage_b,
        mesh=plsc.VectorSubcoreMesh(core_axis_name="c", subcore_axis_name="s"),
        compiler_params=pltpu.CompilerParams(use_tc_tiling_on_sc=False,
                                             needs_layout_passes=False),
        out_type=jax.ShapeDtypeStruct((_T, _NH, _HD, _NP), f32),
        scratch_types=[
            pltpu.VMEM((_HD, _NP), f32),
            pltpu.VMEM((4 * _K, _BLK), f32),
            pltpu.VMEM((4 * _K, _BLK), i32),
            pltpu.VMEM((_HD, _BLK), f32),
        ],
    )
    part = sc_call(w00, w01, w10, w11, idx00, idx01, idx10, idx11, vt)

    part3 = part.reshape(_T, _C, _NP)
    q = query_feat.reshape(_C, _NP)
    yc = pl.pallas_call(
        _stage_c,
        grid=(_NT,),
        in_specs=[
            pl.BlockSpec((_T, _C, _TILE), lambda i: (0, 0, i)),
            pl.BlockSpec((_C, _TILE), lambda i: (0, i)),
            _full((_C, _C)), _full((_C, 1)),
        ],
        out_specs=pl.BlockSpec((_C, _TILE), lambda i: (0, i)),
        out_shape=jax.ShapeDtypeStruct((_C, _NP), f32),
    )(part3, q, out_w.reshape(_C, _C), out_b.reshape(_C, 1))

    return yc.reshape(_B, _C, _H, _W)
